# baseline (speedup 1.0000x reference)
import functools

import jax
import jax.numpy as jnp
from jax import lax
from jax.experimental import pallas as pl
from jax.experimental.pallas import tpu as pltpu
from jax.experimental.pallas import tpu_sc as plsc

NUM_FIELDS = 26
FIELD_DIM = 100000
EMBED_DIM = 16
BATCH = 16384

NC = 2
NS = 16
NW = NC * NS
ITEMS_PER_W = BATCH // NW
CH = 64
NCHUNKS = ITEMS_PER_W // CH
ROWS_PER_CHUNK = CH * NUM_FIELDS
IDX_MINOR = 128
NGATHER = ROWS_PER_CHUNK // IDX_MINOR
IDX_ROWS = BATCH * NUM_FIELDS // IDX_MINOR
IDX_ROWS_PER_W = ITEMS_PER_W * NUM_FIELDS // IDX_MINOR
GRP = 16
NGRP = CH // GRP

_mesh = plsc.VectorSubcoreMesh(core_axis_name="c", subcore_axis_name="s")


@functools.partial(
    pl.kernel,
    mesh=_mesh,
    compiler_params=pltpu.CompilerParams(
        needs_layout_passes=False, use_tc_tiling_on_sc=False
    ),
    out_type=jax.ShapeDtypeStruct((BATCH,), jnp.float32),
    scratch_types=[
        pltpu.VMEM((IDX_ROWS_PER_W, IDX_MINOR), jnp.int32),
        pltpu.VMEM((ROWS_PER_CHUNK, EMBED_DIM), jnp.float32),
        pltpu.VMEM((CH,), jnp.float32),
        pltpu.VMEM((GRP * EMBED_DIM,), jnp.float32),
        pltpu.SemaphoreType.DMA,
    ],
)
def _fm_kernel(idx_hbm, table_hbm, out_hbm, idx_v, rows_v, res_v, tbuf, sem):
    w = lax.axis_index("s") * NC + lax.axis_index("c")
    lane16 = lax.iota(jnp.int32, 16) * EMBED_DIM
    pltpu.sync_copy(idx_hbm.at[pl.ds(w * IDX_ROWS_PER_W, IDX_ROWS_PER_W)], idx_v)

    def chunk_body(c, carry):
        copies = [
            pltpu.async_copy(
                table_hbm.at[idx_v.at[c * NGATHER + j]],
                rows_v.at[pl.ds(j * IDX_MINOR, IDX_MINOR)],
                sem,
            )
            for j in range(NGATHER)
        ]
        for cp in copies:
            cp.wait()

        def grp_body(g, carry2):
            for ii in range(GRP):
                r0 = (g * GRP + ii) * NUM_FIELDS
                acc = jnp.zeros((EMBED_DIM,), jnp.float32)
                acc2 = jnp.zeros((EMBED_DIM,), jnp.float32)
                for f in range(NUM_FIELDS):
                    v = rows_v[r0 + f, :]
                    acc = acc + v
                    acc2 = acc2 + v * v
                tbuf[pl.ds(ii * EMBED_DIM, EMBED_DIM)] = acc * acc - acc2
            tot = jnp.zeros((GRP,), jnp.float32)
            for d in range(EMBED_DIM):
                tot = tot + plsc.load_gather(tbuf, [lane16 + d])
            res_v[pl.ds(g * GRP, GRP)] = 0.5 * tot
            return carry2

        lax.fori_loop(0, NGRP, grp_body, 0)
        out_base = w * ITEMS_PER_W + c * CH
        pltpu.sync_copy(res_v, out_hbm.at[pl.ds(out_base, CH)])
        return carry

    lax.fori_loop(0, NCHUNKS, chunk_body, 0)


def kernel(x, table):
    offsets = jnp.arange(NUM_FIELDS, dtype=jnp.int32) * FIELD_DIM
    idx = (x + offsets[None, :]).reshape(IDX_ROWS, IDX_MINOR)
    out = _fm_kernel(idx, table)
    return out.reshape(BATCH, 1)


# speedup vs baseline: 5.0420x; 5.0420x over previous
import functools

import jax
import jax.numpy as jnp
from jax import lax
from jax.experimental import pallas as pl
from jax.experimental.pallas import tpu as pltpu
from jax.experimental.pallas import tpu_sc as plsc

NUM_FIELDS = 26
FIELD_DIM = 100000
EMBED_DIM = 16
BATCH = 16384
TOTAL_ROWS = NUM_FIELDS * FIELD_DIM

NC = 2
NS = 16
NW = NC * NS

_mesh = plsc.VectorSubcoreMesh(core_axis_name="c", subcore_axis_name="s")

SBW = 1024
NSB_FULL = (TOTAL_ROWS - 64) // SBW
HALF_SB = NSB_FULL // 2
SB_PER_SUB = 80
TAIL_START = NSB_FULL * SBW
TAIL_W = TOTAL_ROWS - TAIL_START
SB_WORDS = SBW * EMBED_DIM
RP_WORDS = TOTAL_ROWS * EMBED_DIM


@functools.partial(
    pl.kernel,
    mesh=_mesh,
    compiler_params=pltpu.CompilerParams(needs_layout_passes=False),
    out_type=jax.ShapeDtypeStruct((RP_WORDS,), jnp.float32),
    scratch_types=[
        pltpu.VMEM((EMBED_DIM, SBW), jnp.float32),
        pltpu.VMEM((EMBED_DIM, SBW), jnp.float32),
        pltpu.VMEM((SB_WORDS,), jnp.float32),
        pltpu.VMEM((SB_WORDS,), jnp.float32),
        pltpu.VMEM((EMBED_DIM, TAIL_W), jnp.float32),
        pltpu.VMEM((TAIL_W * EMBED_DIM,), jnp.float32),
        pltpu.SemaphoreType.DMA,
        pltpu.SemaphoreType.DMA,
    ],
)
def _repack(tT_hbm, rp_out, tin0, tin1, tout0, tout1, tailin, tailout,
            in_sem, out_sem):
    c = lax.axis_index("c")
    s = lax.axis_index("s")
    lane16 = lax.iota(jnp.int32, 16) * EMBED_DIM
    tins = (tin0, tin1)
    touts = (tout0, tout1)

    base = c * HALF_SB + s * SB_PER_SUB
    hi = HALF_SB + c * (NSB_FULL - 1 - HALF_SB)

    def sb(g):
        return jnp.minimum(base + g, hi)

    def fire_in(g, p):
        for a in (0, 1):
            pltpu.async_copy(
                tT_hbm.at[pl.ds(a * 8, 8), pl.ds(sb(g) * SBW, SBW)],
                tins[p].at[pl.ds(a * 8, 8), :],
                in_sem,
            )

    def wait_in(p):
        for a in (0, 1):
            pltpu.make_async_copy(
                tT_hbm.at[pl.ds(a * 8, 8), pl.ds(0, SBW)],
                tins[p].at[pl.ds(a * 8, 8), :],
                in_sem,
            ).wait()

    def fire_out(g, p):
        pltpu.async_copy(
            touts[p], rp_out.at[pl.ds(sb(g) * SB_WORDS, SB_WORDS)], out_sem
        )

    def wait_out(g, p):
        pltpu.make_async_copy(
            touts[p], rp_out.at[pl.ds(sb(g) * SB_WORDS, SB_WORDS)], out_sem
        ).wait()

    def transpose_sb(src, dst, width):
        @plsc.parallel_loop(0, width // 16, unroll=2)
        def l_body(l0):
            ivec = lane16 + l0 * 256
            for d in range(EMBED_DIM):
                v = src[d, pl.ds(l0 * 16, 16)]
                plsc.store_scatter(dst, [ivec + d], v)

    fire_in(0, 0)

    def a_body(k, carry):
        for p in (0, 1):
            g = 2 * k + p
            fire_in(g + 1, 1 - p)
            wait_in(p)

            @pl.when(g >= 2)
            def _():
                wait_out(g - 2, p)

            transpose_sb(tins[p], touts[p], SBW)
            fire_out(g, p)
        return carry

    lax.fori_loop(0, SB_PER_SUB // 2, a_body, 0)
    wait_in(0)
    wait_out(SB_PER_SUB - 2, 0)
    wait_out(SB_PER_SUB - 1, 1)

    @pl.when((c == 1) & (s == NS - 1))
    def _():
        pltpu.sync_copy(tT_hbm.at[:, pl.ds(TAIL_START, TAIL_W)], tailin)
        transpose_sb(tailin, tailout, TAIL_W)
        pltpu.sync_copy(
            tailout, rp_out.at[pl.ds(TAIL_START * EMBED_DIM, TAIL_W * EMBED_DIM)]
        )


ITEMS_PER_W = BATCH // NW
CH = 64
NCHUNKS = ITEMS_PER_W // CH
ROWS_PER_CHUNK = CH * NUM_FIELDS
IDX_MINOR = 128
NGATHER = ROWS_PER_CHUNK // IDX_MINOR
IDX_ROWS = BATCH * NUM_FIELDS // IDX_MINOR
IDX_ROWS_PER_W = ITEMS_PER_W * NUM_FIELDS // IDX_MINOR
GRP = 16
NGRP = CH // GRP


@functools.partial(
    pl.kernel,
    mesh=_mesh,
    compiler_params=pltpu.CompilerParams(
        needs_layout_passes=False, use_tc_tiling_on_sc=False
    ),
    out_type=jax.ShapeDtypeStruct((BATCH,), jnp.float32),
    scratch_types=[
        pltpu.VMEM((IDX_ROWS_PER_W, IDX_MINOR), jnp.int32),
        pltpu.VMEM((ROWS_PER_CHUNK, EMBED_DIM), jnp.float32),
        pltpu.VMEM((ROWS_PER_CHUNK, EMBED_DIM), jnp.float32),
        pltpu.VMEM((ITEMS_PER_W,), jnp.float32),
        pltpu.VMEM((GRP * EMBED_DIM,), jnp.float32),
        pltpu.SemaphoreType.DMA,
    ],
)
def _fm_gather(idx_hbm, table_hbm, out_hbm, idx_v, rows_v0, rows_v1, res_v,
               tbuf, sem):
    w = lax.axis_index("s") * NC + lax.axis_index("c")
    lane16 = lax.iota(jnp.int32, 16) * EMBED_DIM
    rows = (rows_v0, rows_v1)
    pltpu.sync_copy(idx_hbm.at[pl.ds(w * IDX_ROWS_PER_W, IDX_ROWS_PER_W)], idx_v)

    def fire_chunk(ch, p):
        for j in range(NGATHER):
            pltpu.async_copy(
                table_hbm.at[idx_v.at[ch * NGATHER + j]],
                rows[p].at[pl.ds(j * IDX_MINOR, IDX_MINOR)],
                sem,
            )

    def wait_chunk(p):
        for j in range(NGATHER):
            pltpu.make_async_copy(
                table_hbm.at[idx_v.at[0]],
                rows[p].at[pl.ds(j * IDX_MINOR, IDX_MINOR)],
                sem,
            ).wait()

    def compute_chunk(ch, p):
        def grp_body(g, carry2):
            for ii in range(GRP):
                r0 = (g * GRP + ii) * NUM_FIELDS
                acc = jnp.zeros((EMBED_DIM,), jnp.float32)
                acc2 = jnp.zeros((EMBED_DIM,), jnp.float32)
                for f in range(NUM_FIELDS):
                    v = rows[p][r0 + f, :]
                    acc = acc + v
                    acc2 = acc2 + v * v
                tbuf[pl.ds(ii * EMBED_DIM, EMBED_DIM)] = acc * acc - acc2
            tot = jnp.zeros((GRP,), jnp.float32)
            for d in range(EMBED_DIM):
                tot = tot + plsc.load_gather(tbuf, [lane16 + d])
            res_v[pl.ds(ch * CH + g * GRP, GRP)] = 0.5 * tot
            return carry2

        lax.fori_loop(0, NGRP, grp_body, 0)

    fire_chunk(0, 0)

    def chunk_body(k, carry):
        for p in (0, 1):
            ch = 2 * k + p

            @pl.when(ch + 1 < NCHUNKS)
            def _():
                fire_chunk(ch + 1, 1 - p)

            wait_chunk(p)
            compute_chunk(ch, p)
        return carry

    lax.fori_loop(0, NCHUNKS // 2, chunk_body, 0)
    pltpu.sync_copy(res_v, out_hbm.at[pl.ds(w * ITEMS_PER_W, ITEMS_PER_W)])


def kernel(x, table):
    offsets = jnp.arange(NUM_FIELDS, dtype=jnp.int32) * FIELD_DIM
    idx = (x + offsets[None, :]).reshape(IDX_ROWS, IDX_MINOR)
    rp = _repack(table.T)
    tpacked = rp.reshape(TOTAL_ROWS, EMBED_DIM)
    out = _fm_gather(idx, tpacked)
    return out.reshape(BATCH, 1)
